# baseline (speedup 1.0000x reference)
import functools

import jax
import jax.numpy as jnp
import ml_dtypes
import numpy as np
from jax import lax
from jax.experimental import pallas as pl
from jax.experimental.pallas import tpu as pltpu
from jax.experimental.pallas import tpu_sc as plsc

SR = 22050
N_FFT = 1024
HOP = 256
AUDIO_DURATION = 30
FIXED_LEN = AUDIO_DURATION * SR // HOP + 1
N_FRAMES = 1 + AUDIO_DURATION * SR // HOP
NK = N_FFT // 2
NF = 88

_BLK_F = 152
_GRID = N_FRAMES // _BLK_F

_ROWS = 2688
_TBL = NF * _ROWS
_NPTS = N_FRAMES * NK
_NCORES = 2
_NSUB = 16
_NW = _NCORES * _NSUB
_NPW = _NPTS // _NW
_TSLICE = _TBL // _NSUB


def _hann_periodic(n):
    k = np.arange(n)
    return (0.5 - 0.5 * np.cos(2.0 * np.pi * k / n)).astype(np.float32)


def _cyc_grad(w):
    wp = np.concatenate([w[-1:], w, w[:1]])
    return np.gradient(wp)[1:-1].astype(np.float32)


def _build_consts():
    win = _hann_periodic(N_FFT).astype(np.float64)
    dwin = _cyc_grad(_hann_periodic(N_FFT)).astype(np.float64)
    t = np.arange(N_FFT, dtype=np.float64)[:, None]
    k = np.arange(NK, dtype=np.float64)[None, :]
    ang = 2.0 * np.pi * t * k / N_FFT
    c, s = np.cos(ang), -np.sin(ang)
    mats = (win[:, None] * c, win[:, None] * s,
            dwin[:, None] * c, dwin[:, None] * s)
    split = []
    for m in mats:
        m32 = m.astype(np.float32)
        hi = m32.astype(ml_dtypes.bfloat16)
        lo = (m32 - hi.astype(np.float32)).astype(ml_dtypes.bfloat16)
        split.append((hi, lo))

    ratio = 1.059463094
    lowest = 27.5
    hz = [lowest * ratio ** i for i in range(89)]
    fb = np.array([(x + y) / 2.0 for x, y in zip([lowest / ratio] + hz, hz)],
                  dtype=np.float64).astype(np.float32)
    a_edge = lowest * (1 + ratio) / (2 * ratio)
    c1 = float(np.float32(1.0 / np.log(ratio)))
    c2 = float(np.float32(np.log(a_edge) / np.log(ratio)))

    tb = np.arange(0.0, AUDIO_DURATION, HOP / SR).astype(np.float32)
    nt = tb.size - 1
    ft = (np.arange(N_FRAMES, dtype=np.float32) * np.float32(HOP / SR))
    ti = np.searchsorted(tb, ft, side="right") - 1
    ti = np.where(ft == tb[-1], nt - 1, ti)
    n = np.arange(N_FRAMES)
    assert np.all((ti == n) | (ti == n - 1))
    ti_col = ti.astype(np.int32)[:, None]
    return tuple(split), fb, c1, c2, ti_col, nt


_MATS, _FB, _C1, _C2, _TI, _NT = _build_consts()
(_CHR_H, _CHR_L), (_CHI_H, _CHI_L), (_CDR_H, _CDR_L), (_CDI_H, _CDI_L) = _MATS
_FB0 = float(_FB[0])
_FB88 = float(_FB[NF])
_FREQ_STEP = float(np.float32(SR / N_FFT))
_CORR_SCALE = float(np.float32(0.5 * SR / np.pi))
_ZEROS_TBL = np.zeros((_TBL,), np.float32)


def _point_body(z0, z1, z2, z3, tirow, chrh, chrl, chih, chil,
                cdrh, cdrl, cdih, cdil, idx_out, val_out):
    frames = jnp.concatenate(
        [z0[...], z1[...], z2[...], z3[...]], axis=1)
    frames = jnp.clip(frames, -1.0, 1.0)
    fh = frames.astype(jnp.bfloat16)
    fl = (frames - fh.astype(jnp.float32)).astype(jnp.bfloat16)

    def mm3(mh, ml):
        d = functools.partial(jnp.dot, preferred_element_type=jnp.float32)
        return d(fh, mh[...]) + (d(fh, ml[...]) + d(fl, mh[...]))

    hr = mm3(chrh, chrl)
    hi = mm3(chih, chil)
    dr = mm3(cdrh, cdrl)
    di = mm3(cdih, cdil)
    mag2 = hr * hr + hi * hi
    w = jnp.sqrt(mag2)
    corr = -(di * hr - dr * hi) / jnp.maximum(mag2, 1e-30) * _CORR_SCALE
    base = jax.lax.broadcasted_iota(
        jnp.int32, (1, NK), 1).astype(jnp.float32) * _FREQ_STEP
    f = jnp.where(mag2 > 0, base + corr, base)

    fi = jnp.floor(jnp.log(jnp.maximum(f, 1e-3)) * _C1 - _C2)
    fi = jnp.clip(fi, 0.0, 87.0).astype(jnp.int32)
    valid = (f >= _FB0) & (f <= _FB88)
    idx_out[...] = fi * _ROWS + tirow[...]
    val_out[...] = jnp.where(valid, w, 0.0)


def _make_sc_scatter(npts):
    npw = npts // _NW
    assert npw * _NW == npts and npw % 8 == 0 and npw >= _TSLICE

    @functools.partial(
        pl.kernel,
        mesh=plsc.VectorSubcoreMesh(core_axis_name="c", subcore_axis_name="s"),
        out_type=jax.ShapeDtypeStruct((_NCORES * _TBL,), jnp.float32),
        scratch_types=[
            pltpu.VMEM((npw,), jnp.int32),
            pltpu.VMEM((npw,), jnp.float32),
            pltpu.VMEM_SHARED((_TBL,), jnp.float32),
            pltpu.SemaphoreType.DMA,
        ],
    )
    def sc_scatter(idx_hbm, val_hbm, zeros_hbm, out_hbm,
                   idx_v, val_v, tbl, sem):
        c = lax.axis_index("c")
        s = lax.axis_index("s")
        wid = s * _NCORES + c
        stage = val_v.at[pl.ds(0, _TSLICE)]
        pltpu.sync_copy(zeros_hbm.at[pl.ds(s * _TSLICE, _TSLICE)], stage)
        pltpu.sync_copy(stage, tbl.at[pl.ds(s * _TSLICE, _TSLICE)])
        plsc.subcore_barrier()
        base = wid * npw
        pltpu.sync_copy(idx_hbm.at[pl.ds(base, npw)], idx_v)
        pltpu.sync_copy(val_hbm.at[pl.ds(base, npw)], val_v)
        pltpu.sync_copy(val_v, tbl.at[idx_v], add=True)
        plsc.subcore_barrier()
        pltpu.sync_copy(tbl.at[pl.ds(s * _TSLICE, _TSLICE)], stage)
        pltpu.sync_copy(stage,
                        out_hbm.at[pl.ds(c * _TBL + s * _TSLICE, _TSLICE)])

    return sc_scatter


_F_HALF1 = 1368
_F_HALF2 = N_FRAMES - _F_HALF1
_SC_SCATTER1 = _make_sc_scatter(_F_HALF1 * NK)
_SC_SCATTER2 = _make_sc_scatter(_F_HALF2 * NK)


def _merge_body(t1, t2, out):
    x1 = t1[...]
    x2 = t2[...]
    h = x1[0] + x1[1] + x2[0] + x2[1]
    m = h[:, 0:_NT]
    scale = 1.0 / jnp.maximum(jnp.max(m), 1e-12)
    out[...] = jnp.concatenate([m, m[:, _NT - 1:_NT]], axis=1) * scale


@jax.jit
def kernel(waveform):
    pad = N_FFT // 2
    ypad = jnp.pad(waveform, (pad, pad + 4))
    z = ypad.reshape(-1, 256)
    z0 = z[0:N_FRAMES]
    z1 = z[1:N_FRAMES + 1]
    z2 = z[2:N_FRAMES + 2]
    z3 = z[3:N_FRAMES + 3]

    zspec = pl.BlockSpec((_BLK_F, 256), lambda i: (i, 0))
    tspec = pl.BlockSpec((_BLK_F, 1), lambda i: (i, 0))
    mspec = pl.BlockSpec((N_FFT, NK), lambda i: (0, 0))

    def tc_points(lo, nf_half):
        return pl.pallas_call(
            _point_body,
            grid=(nf_half // _BLK_F,),
            in_specs=[zspec, zspec, zspec, zspec, tspec] + [mspec] * 8,
            out_specs=[pl.BlockSpec((_BLK_F, NK), lambda i: (i, 0)),
                       pl.BlockSpec((_BLK_F, NK), lambda i: (i, 0))],
            out_shape=[jax.ShapeDtypeStruct((nf_half, NK), jnp.int32),
                       jax.ShapeDtypeStruct((nf_half, NK), jnp.float32)],
        )(z0[lo:lo + nf_half], z1[lo:lo + nf_half], z2[lo:lo + nf_half],
          z3[lo:lo + nf_half], _TI[lo:lo + nf_half],
          _CHR_H, _CHR_L, _CHI_H, _CHI_L, _CDR_H, _CDR_L, _CDI_H, _CDI_L)

    idx1, val1 = tc_points(0, _F_HALF1)
    tbl1 = _SC_SCATTER1(idx1.reshape(-1), val1.reshape(-1), _ZEROS_TBL)
    idx2, val2 = tc_points(_F_HALF1, _F_HALF2)
    tbl2 = _SC_SCATTER2(idx2.reshape(-1), val2.reshape(-1), _ZEROS_TBL)

    out = pl.pallas_call(
        _merge_body,
        in_specs=[pl.BlockSpec((_NCORES, NF, _ROWS), lambda: (0, 0, 0)),
                  pl.BlockSpec((_NCORES, NF, _ROWS), lambda: (0, 0, 0))],
        out_specs=pl.BlockSpec((NF, FIXED_LEN), lambda: (0, 0)),
        out_shape=jax.ShapeDtypeStruct((NF, FIXED_LEN), jnp.float32),
    )(tbl1.reshape(_NCORES, NF, _ROWS), tbl2.reshape(_NCORES, NF, _ROWS))
    return out


# speedup vs baseline: 1.2076x; 1.2076x over previous
import functools

import jax
import jax.numpy as jnp
import ml_dtypes
import numpy as np
from jax.experimental import pallas as pl

SR = 22050
N_FFT = 1024
HOP = 256
AUDIO_DURATION = 30
FIXED_LEN = AUDIO_DURATION * SR // HOP + 1
N_FRAMES = 1 + AUDIO_DURATION * SR // HOP
NK = N_FFT // 2
NF = 88

_BLK_F = 256
_NPF = 2816
_GRID = _NPF // _BLK_F


def _hann_periodic(n):
    k = np.arange(n)
    return (0.5 - 0.5 * np.cos(2.0 * np.pi * k / n)).astype(np.float32)


def _cyc_grad(w):
    wp = np.concatenate([w[-1:], w, w[:1]])
    return np.gradient(wp)[1:-1].astype(np.float32)


def _build_consts():
    win = _hann_periodic(N_FFT).astype(np.float64)
    dwin = _cyc_grad(_hann_periodic(N_FFT)).astype(np.float64)
    t = np.arange(N_FFT, dtype=np.float64)[:, None]
    k = np.arange(NK, dtype=np.float64)[None, :]
    ang = 2.0 * np.pi * t * k / N_FFT
    c, s = np.cos(ang), -np.sin(ang)
    mats = (win[:, None] * c, win[:, None] * s,
            dwin[:, None] * c, dwin[:, None] * s)
    split = []
    for m in mats:
        m32 = m.astype(np.float32)
        hi = m32.astype(ml_dtypes.bfloat16)
        lo = (m32 - hi.astype(np.float32)).astype(ml_dtypes.bfloat16)
        split.append((hi, lo))

    ratio = 1.059463094
    lowest = 27.5
    hz = [lowest * ratio ** i for i in range(89)]
    fb = np.array([(x + y) / 2.0 for x, y in zip([lowest / ratio] + hz, hz)],
                  dtype=np.float64).astype(np.float32)

    tb = np.arange(0.0, AUDIO_DURATION, HOP / SR).astype(np.float32)
    nt = tb.size - 1
    ft = (np.arange(N_FRAMES, dtype=np.float32) * np.float32(HOP / SR))
    ti = np.searchsorted(tb, ft, side="right") - 1
    ti = np.where(ft == tb[-1], nt - 1, ti)
    n = np.arange(N_FRAMES)
    assert np.all((ti == n) | (ti == n - 1))
    e0 = (ti[:nt] == np.arange(nt)).astype(np.float32)[:, None]
    e1 = (ti[1:nt + 1] == np.arange(nt)).astype(np.float32)[:, None]
    return tuple(split), fb, e0, e1, nt


_MATS, _FB, _E0, _E1, _NT = _build_consts()
(_CHR_H, _CHR_L), (_CHI_H, _CHI_L), (_CDR_H, _CDR_L), (_CDI_H, _CDI_L) = _MATS
_EDGES = [float(x) for x in _FB]
_FREQ_STEP = float(np.float32(SR / N_FFT))
_CORR_SCALE = float(np.float32(0.5 * SR / np.pi))


def _hist_body(z0, z1, z2, z3, chrh, chrl, chih, chil,
               cdrh, cdrl, cdih, cdil, out):
    frames = jnp.concatenate(
        [z0[...], z1[...], z2[...], z3[...]], axis=1)
    frames = jnp.clip(frames, -1.0, 1.0)
    fh = frames.astype(jnp.bfloat16)
    fl = (frames - fh.astype(jnp.float32)).astype(jnp.bfloat16)

    def mm3(mh, ml):
        d = functools.partial(jnp.dot, preferred_element_type=jnp.float32)
        return d(fh, mh[...]) + (d(fh, ml[...]) + d(fl, mh[...]))

    hr = mm3(chrh, chrl)
    hi = mm3(chih, chil)
    dr = mm3(cdrh, cdrl)
    di = mm3(cdih, cdil)
    mag2 = hr * hr + hi * hi
    w = jnp.sqrt(mag2)
    corr = -(di * hr - dr * hi) / jnp.maximum(mag2, 1e-30) * _CORR_SCALE
    base = jax.lax.broadcasted_iota(
        jnp.int32, (1, NK), 1).astype(jnp.float32) * _FREQ_STEP
    f = jnp.where(mag2 > 0, base + corr, base)

    cols = []
    for b in range(NF + 1):
        if b < NF:
            m = f >= _EDGES[b]
        else:
            m = f > _EDGES[NF]
        cols.append(jnp.sum(jnp.where(m, w, 0.0), axis=1, keepdims=True))
    g = jnp.concatenate(cols, axis=1)
    out[...] = g[:, :NF] - g[:, 1:NF + 1]


def _merge_body(hf, e0, e1, out):
    h = hf[...]
    merged = h[0:_NT] * e0[...] + h[1:_NT + 1] * e1[...]
    scale = 1.0 / jnp.maximum(jnp.max(merged), 1e-12)
    mp = jnp.concatenate([merged, merged[_NT - 1:_NT]], axis=0) * scale
    out[...] = mp.T


@jax.jit
def kernel(waveform):
    pad = N_FFT // 2
    total = (_NPF + 4) * 256
    ypad = jnp.pad(waveform, (pad, total - waveform.shape[0] - pad))
    z = ypad.reshape(-1, 256)
    z0 = z[0:_NPF]
    z1 = z[1:_NPF + 1]
    z2 = z[2:_NPF + 2]
    z3 = z[3:_NPF + 3]

    zspec = pl.BlockSpec((_BLK_F, 256), lambda i: (i, 0))
    mspec = pl.BlockSpec((N_FFT, NK), lambda i: (0, 0))
    hf = pl.pallas_call(
        _hist_body,
        grid=(_GRID,),
        in_specs=[zspec, zspec, zspec, zspec] + [mspec] * 8,
        out_specs=pl.BlockSpec((_BLK_F, NF), lambda i: (i, 0)),
        out_shape=jax.ShapeDtypeStruct((_NPF, NF), jnp.float32),
    )(z0, z1, z2, z3, _CHR_H, _CHR_L, _CHI_H, _CHI_L,
      _CDR_H, _CDR_L, _CDI_H, _CDI_L)

    out = pl.pallas_call(
        _merge_body,
        in_specs=[pl.BlockSpec((_NPF, NF), lambda: (0, 0)),
                  pl.BlockSpec((_NT, 1), lambda: (0, 0)),
                  pl.BlockSpec((_NT, 1), lambda: (0, 0))],
        out_specs=pl.BlockSpec((NF, FIXED_LEN), lambda: (0, 0)),
        out_shape=jax.ShapeDtypeStruct((NF, FIXED_LEN), jnp.float32),
    )(hf, _E0, _E1)
    return out


# speedup vs baseline: 1.2713x; 1.0528x over previous
import functools

import jax
import jax.numpy as jnp
import ml_dtypes
import numpy as np
from jax.experimental import pallas as pl

SR = 22050
N_FFT = 1024
HOP = 256
AUDIO_DURATION = 30
FIXED_LEN = AUDIO_DURATION * SR // HOP + 1
N_FRAMES = 1 + AUDIO_DURATION * SR // HOP
NK = N_FFT // 2
NF = 88

_BLK_F = 152
_GRID = N_FRAMES // _BLK_F


def _hann_periodic(n):
    k = np.arange(n)
    return (0.5 - 0.5 * np.cos(2.0 * np.pi * k / n)).astype(np.float32)


def _cyc_grad(w):
    wp = np.concatenate([w[-1:], w, w[:1]])
    return np.gradient(wp)[1:-1].astype(np.float32)


def _build_consts():
    win = _hann_periodic(N_FFT).astype(np.float64)
    dwin = _cyc_grad(_hann_periodic(N_FFT)).astype(np.float64)
    t = np.arange(N_FFT, dtype=np.float64)[:, None]
    k = np.arange(NK, dtype=np.float64)[None, :]
    ang = 2.0 * np.pi * t * k / N_FFT
    c, s = np.cos(ang), -np.sin(ang)
    mats = (win[:, None] * c, win[:, None] * s,
            dwin[:, None] * c, dwin[:, None] * s)
    split = []
    for m in mats:
        m32 = m.astype(np.float32)
        hi = m32.astype(ml_dtypes.bfloat16)
        lo = (m32 - hi.astype(np.float32)).astype(ml_dtypes.bfloat16)
        split.append((hi, lo))

    ratio = 1.059463094
    lowest = 27.5
    hz = [lowest * ratio ** i for i in range(89)]
    fb = np.array([(x + y) / 2.0 for x, y in zip([lowest / ratio] + hz, hz)],
                  dtype=np.float64).astype(np.float32)

    tb = np.arange(0.0, AUDIO_DURATION, HOP / SR).astype(np.float32)
    nt = tb.size - 1
    ft = (np.arange(N_FRAMES, dtype=np.float32) * np.float32(HOP / SR))
    ti = np.searchsorted(tb, ft, side="right") - 1
    ti = np.where(ft == tb[-1], nt - 1, ti)
    n = np.arange(N_FRAMES)
    assert np.all((ti == n) | (ti == n - 1))
    e0 = (ti[:nt] == np.arange(nt)).astype(np.float32)[:, None]
    e1 = (ti[1:nt + 1] == np.arange(nt)).astype(np.float32)[:, None]
    return tuple(split), fb, e0, e1, nt


_MATS, _FB, _E0, _E1, _NT = _build_consts()
(_CHR_H, _CHR_L), (_CHI_H, _CHI_L), (_CDR_H, _CDR_L), (_CDI_H, _CDI_L) = _MATS
_EDGES = [float(x) for x in _FB]
_FREQ_STEP = float(np.float32(SR / N_FFT))
_CORR_SCALE = float(np.float32(0.5 * SR / np.pi))


def _hist_body(z0, z1, z2, z3, chrh, chrl, chih, chil,
               cdrh, cdrl, cdih, cdil, out):
    frames = jnp.concatenate(
        [z0[...], z1[...], z2[...], z3[...]], axis=1)
    frames = jnp.clip(frames, -1.0, 1.0)
    fh = frames.astype(jnp.bfloat16)
    fl = (frames - fh.astype(jnp.float32)).astype(jnp.bfloat16)

    def mm3(mh, ml):
        d = functools.partial(jnp.dot, preferred_element_type=jnp.float32)
        return d(fh, mh[...]) + (d(fh, ml[...]) + d(fl, mh[...]))

    hr = mm3(chrh, chrl)
    hi = mm3(chih, chil)
    dr = mm3(cdrh, cdrl)
    di = mm3(cdih, cdil)
    mag2 = hr * hr + hi * hi
    w = jnp.sqrt(mag2)
    corr = -(di * hr - dr * hi) / jnp.maximum(mag2, 1e-30) * _CORR_SCALE
    base = jax.lax.broadcasted_iota(
        jnp.int32, (1, NK), 1).astype(jnp.float32) * _FREQ_STEP
    f = jnp.where(mag2 > 0, base + corr, base)

    cols = []
    for b in range(NF + 1):
        if b < NF:
            m = f >= _EDGES[b]
        else:
            m = f > _EDGES[NF]
        cols.append(jnp.sum(jnp.where(m, w, 0.0), axis=1, keepdims=True))
    g = jnp.concatenate(cols, axis=1)
    out[...] = g[:, :NF] - g[:, 1:NF + 1]


def _merge_body(hf, e0, e1, out):
    h = hf[...]
    merged = h[0:_NT] * e0[...] + h[1:_NT + 1] * e1[...]
    scale = 1.0 / jnp.maximum(jnp.max(merged), 1e-12)
    mp = jnp.concatenate([merged, merged[_NT - 1:_NT]], axis=0) * scale
    out[...] = mp.T


@jax.jit
def kernel(waveform):
    pad = N_FFT // 2
    ypad = jnp.pad(waveform, (pad, pad + 4))
    z = ypad.reshape(-1, 256)
    z0 = z[0:N_FRAMES]
    z1 = z[1:N_FRAMES + 1]
    z2 = z[2:N_FRAMES + 2]
    z3 = z[3:N_FRAMES + 3]

    zspec = pl.BlockSpec((_BLK_F, 256), lambda i: (i, 0))
    mspec = pl.BlockSpec((N_FFT, NK), lambda i: (0, 0))
    hf = pl.pallas_call(
        _hist_body,
        grid=(_GRID,),
        in_specs=[zspec, zspec, zspec, zspec] + [mspec] * 8,
        out_specs=pl.BlockSpec((_BLK_F, NF), lambda i: (i, 0)),
        out_shape=jax.ShapeDtypeStruct((N_FRAMES, NF), jnp.float32),
    )(z0, z1, z2, z3, _CHR_H, _CHR_L, _CHI_H, _CHI_L,
      _CDR_H, _CDR_L, _CDI_H, _CDI_L)

    out = pl.pallas_call(
        _merge_body,
        in_specs=[pl.BlockSpec((N_FRAMES, NF), lambda: (0, 0)),
                  pl.BlockSpec((_NT, 1), lambda: (0, 0)),
                  pl.BlockSpec((_NT, 1), lambda: (0, 0))],
        out_specs=pl.BlockSpec((NF, FIXED_LEN), lambda: (0, 0)),
        out_shape=jax.ShapeDtypeStruct((NF, FIXED_LEN), jnp.float32),
    )(hf, _E0, _E1)
    return out
